# Initial kernel scaffold; baseline (speedup 1.0000x reference)
import jax
import jax.numpy as jnp
from jax.experimental import pallas as pl


def kernel(x, edge_index, graph_neigh, all_edge_weights, W1, b1, W2, b2, Wa, ba, qa):
    raise NotImplementedError("write your pallas kernel here")


# speedup vs baseline: 1.0013x; 1.0013x over previous
import functools

import jax
import jax.numpy as jnp
from jax import lax
from jax.experimental import pallas as pl
from jax.experimental.pallas import tpu as pltpu

N = 10000
E = 160000
IN = 128
OUT = 64
HID = 64
G = 2

_BR = 400


def _readout_body(gn_ref, fused_ref, g_ref):
    gn = gn_ref[...]
    fused = fused_ref[...]
    vsum = jax.lax.dot_general(
        gn, fused, (((1,), (0,)), ((), ())),
        preferred_element_type=jnp.float32,
        precision=jax.lax.Precision.HIGHEST,
    )
    row_sum = jnp.sum(gn, axis=1, keepdims=True)
    g = vsum / row_sum
    nrm = jnp.sqrt(jnp.sum(g * g, axis=1, keepdims=True))
    g_ref[...] = g / jnp.maximum(nrm, 1e-12)


def _readout(graph_neigh, fused):
    n = graph_neigh.shape[0]
    d = fused.shape[1]
    return pl.pallas_call(
        _readout_body,
        grid=(n // _BR,),
        in_specs=[
            pl.BlockSpec((_BR, n), lambda i: (i, 0)),
            pl.BlockSpec((n, d), lambda i: (0, 0)),
        ],
        out_specs=pl.BlockSpec((_BR, d), lambda i: (i, 0)),
        out_shape=jax.ShapeDtypeStruct((n, d), jnp.float32),
    )(graph_neigh, fused)


def _gcn(x, src, dst, ew, W, b, num_nodes):
    loop = jnp.arange(num_nodes, dtype=src.dtype)
    src2 = jnp.concatenate([src, loop])
    dst2 = jnp.concatenate([dst, loop])
    ew2 = jnp.concatenate([ew, jnp.ones((num_nodes,), dtype=x.dtype)])
    deg = jax.ops.segment_sum(ew2, dst2, num_segments=num_nodes)
    dinv = jnp.where(deg > 0, 1.0 / jnp.sqrt(deg), 0.0)
    norm = dinv[src2] * ew2 * dinv[dst2]
    xw = x @ W
    msg = xw[src2] * norm[:, None]
    return jax.ops.segment_sum(msg, dst2, num_segments=num_nodes) + b


def kernel(x, edge_index, graph_neigh, all_edge_weights, W1, b1, W2, b2, Wa, ba, qa):
    num_nodes = x.shape[0]
    src, dst = edge_index[0], edge_index[1]
    hiddens = []
    for i in range(G):
        h = _gcn(x, src, dst, all_edge_weights[i], W1, b1, num_nodes)
        hiddens.append(jax.nn.relu(h))
    scores = []
    for i in range(G):
        ha = jnp.tanh(hiddens[i] @ Wa[i] + ba[i])
        scores.append(ha @ qa[i])
    attn_scores = jnp.stack(scores, axis=1)
    attn_weights = jax.nn.softmax(attn_scores, axis=1)
    fused = attn_weights[:, 0:1] * hiddens[0] + attn_weights[:, 1:2] * hiddens[1]
    ones_ew = jnp.ones((src.shape[0],), dtype=x.dtype)
    h2 = _gcn(fused, src, dst, ones_ew, W2, b2, num_nodes)
    emb = jax.nn.relu(h2)
    g = _readout(graph_neigh, fused)
    return (fused, emb, g, hiddens[0], hiddens[1], attn_weights)


# speedup vs baseline: 10.1715x; 10.1582x over previous
import functools

import jax
import jax.numpy as jnp
from jax import lax
from jax.experimental import pallas as pl
from jax.experimental.pallas import tpu as pltpu
from jax.experimental.pallas import tpu_sc as plsc

N = 10000
E = 160000
IN = 128
OUT = 64
HID = 64
G = 2

NP = 10240
CH = 128
EP = 163840
ROWS = EP // CH
NC = 2
NS = 16
RPC = ROWS // NC
RPT = RPC // NS
RPD = ROWS // NS
NPT = NP // NS

_BR = 400


def _zero_vec(ref, nvec):
    zeros = jnp.zeros((16,), jnp.float32)

    def body(i, _):
        ref[pl.ds(i * 16, 16)] = zeros
        return 0

    lax.fori_loop(0, nvec, body, 0)


def _zero_rows(ref, nrows, width):
    zeros = jnp.zeros((16,), jnp.float32)

    def body(r, _):
        for k in range(width // 16):
            ref[r, pl.ds(k * 16, 16)] = zeros
        return 0

    lax.fori_loop(0, nrows, body, 0)


CHK = 8


def _deg_body(dst2d, ew2d, unit2d, outdeg,
              deg0_sh, deg1_sh, deg2_sh, dbuf, vbuf, zbuf, sem):
    cid = lax.axis_index("c")
    sid = lax.axis_index("s")

    _zero_vec(zbuf, NPT // 16)
    pltpu.sync_copy(zbuf, deg0_sh.at[pl.ds(sid * NPT, NPT)])
    pltpu.sync_copy(zbuf, deg1_sh.at[pl.ds(sid * NPT, NPT)])
    pltpu.sync_copy(zbuf, deg2_sh.at[pl.ds(sid * NPT, NPT)])
    plsc.subcore_barrier()

    @pl.when(cid == 0)
    def _():
        def deg_chunk(t, _):
            r0 = sid * RPD + t * CHK
            pltpu.sync_copy(dst2d.at[pl.ds(r0, CHK)], dbuf)
            pltpu.sync_copy(ew2d.at[0, pl.ds(r0, CHK)], vbuf.at[0])
            pltpu.sync_copy(ew2d.at[1, pl.ds(r0, CHK)], vbuf.at[1])
            pltpu.sync_copy(unit2d.at[pl.ds(r0, CHK)], vbuf.at[2])

            def deg_row(j, _):
                c1 = pltpu.async_copy(vbuf.at[0, j], deg0_sh.at[dbuf.at[j]],
                                      sem, add=True)
                c2 = pltpu.async_copy(vbuf.at[1, j], deg1_sh.at[dbuf.at[j]],
                                      sem, add=True)
                c3 = pltpu.async_copy(vbuf.at[2, j], deg2_sh.at[dbuf.at[j]],
                                      sem, add=True)
                c1.wait()
                c2.wait()
                c3.wait()
                return 0

            lax.fori_loop(0, CHK, deg_row, 0)
            return 0

        lax.fori_loop(0, RPD // CHK, deg_chunk, 0)

    plsc.subcore_barrier()

    @pl.when(jnp.logical_and(cid == 0, sid == 0))
    def _():
        pltpu.sync_copy(deg0_sh, outdeg.at[pl.ds(0, NP)])
        pltpu.sync_copy(deg1_sh, outdeg.at[pl.ds(NP, NP)])
        pltpu.sync_copy(deg2_sh, outdeg.at[pl.ds(2 * NP, NP)])


def _deg_sc(dst2d, ew2d, unit2d):
    mesh = plsc.VectorSubcoreMesh(core_axis_name="c", subcore_axis_name="s")
    f = pl.kernel(
        _deg_body,
        out_type=[jax.ShapeDtypeStruct((3 * NP,), jnp.float32)],
        mesh=mesh,
        scratch_types=[
            pltpu.MemorySpace.VMEM_SHARED((NP,), jnp.float32),
            pltpu.MemorySpace.VMEM_SHARED((NP,), jnp.float32),
            pltpu.MemorySpace.VMEM_SHARED((NP,), jnp.float32),
            pltpu.VMEM((CHK, CH), jnp.int32),
            pltpu.VMEM((3, CHK, CH), jnp.float32),
            pltpu.VMEM((NPT,), jnp.float32),
            pltpu.SemaphoreType.DMA,
        ],
    )
    return f(dst2d, ew2d, unit2d)


def _conv1_body(src2d, dst2d, ew2d, xs_hbm, outacc,
                acc_sh, sbuf, dbuf, vbuf, rows, msg, cb0, cb1, sem):
    cid = lax.axis_index("c")
    sid = lax.axis_index("s")

    _zero_rows(msg, CH, IN)
    for t in range(NPT // CH):
        pltpu.sync_copy(msg, acc_sh.at[pl.ds(sid * NPT + t * CH, CH)])
    plsc.subcore_barrier()

    def conv_chunk(t, _):
        r0 = cid * RPC + sid * RPT + t * CHK
        pltpu.sync_copy(src2d.at[pl.ds(r0, CHK)], sbuf)
        pltpu.sync_copy(dst2d.at[pl.ds(r0, CHK)], dbuf)
        pltpu.sync_copy(ew2d.at[0, pl.ds(r0, CHK)], vbuf.at[0])
        pltpu.sync_copy(ew2d.at[1, pl.ds(r0, CHK)], vbuf.at[1])

        def conv_body(j, _):
            pltpu.async_copy(xs_hbm.at[sbuf.at[j]], rows, sem).wait()

            def cb_body(k, _):
                s16 = pl.ds(k * 16, 16)
                cb0[s16] = vbuf[0, j, s16]
                cb1[s16] = vbuf[1, j, s16]
                return 0

            lax.fori_loop(0, CH // 16, cb_body, 0)

            def scale_body(gi, _):
                c0v = cb0[pl.ds(gi * 16, 16)]
                c1v = cb1[pl.ds(gi * 16, 16)]
                for u in range(16):
                    e = gi * 16 + u
                    c0 = c0v[u]
                    c1 = c1v[u]
                    for k in range(OUT // 16):
                        msg[e, pl.ds(k * 16, 16)] = (
                            rows[e, pl.ds(k * 16, 16)] * c0)
                        msg[e, pl.ds(OUT + k * 16, 16)] = (
                            rows[e, pl.ds(OUT + k * 16, 16)] * c1)
                return 0

            lax.fori_loop(0, CH // 16, scale_body, 0)

            pltpu.async_copy(msg, acc_sh.at[dbuf.at[j]], sem, add=True).wait()
            return 0

        lax.fori_loop(0, CHK, conv_body, 0)
        return 0

    lax.fori_loop(0, RPT // CHK, conv_chunk, 0)
    plsc.subcore_barrier()

    @pl.when(sid == 0)
    def _():
        pltpu.sync_copy(acc_sh, outacc.at[cid])


def _conv1_sc(src2d, dst2d, ew2d, xs_pack):
    mesh = plsc.VectorSubcoreMesh(core_axis_name="c", subcore_axis_name="s")
    f = pl.kernel(
        _conv1_body,
        out_type=[jax.ShapeDtypeStruct((NC, NP, IN), jnp.float32)],
        mesh=mesh,
        scratch_types=[
            pltpu.MemorySpace.VMEM_SHARED((NP, IN), jnp.float32),
            pltpu.VMEM((CHK, CH), jnp.int32),
            pltpu.VMEM((CHK, CH), jnp.int32),
            pltpu.VMEM((G, CHK, CH), jnp.float32),
            pltpu.VMEM((CH, IN), jnp.float32),
            pltpu.VMEM((CH, IN), jnp.float32),
            pltpu.VMEM((CH,), jnp.float32),
            pltpu.VMEM((CH,), jnp.float32),
            pltpu.SemaphoreType.DMA,
        ],
    )
    return f(src2d, dst2d, ew2d, xs_pack)


def _conv2_body(src2d, dst2d, xs2_hbm, out2,
                acc_sh, srcC, dstC, rows, sem):
    cid = lax.axis_index("c")
    sid = lax.axis_index("s")

    _zero_rows(rows, CH, IN)
    for t in range(NPT // CH):
        pltpu.sync_copy(rows, acc_sh.at[pl.ds(sid * NPT + t * CH, CH)])
    plsc.subcore_barrier()

    base = cid * RPC + sid * RPT
    pltpu.sync_copy(src2d.at[pl.ds(base, RPT)], srcC)
    pltpu.sync_copy(dst2d.at[pl.ds(base, RPT)], dstC)

    def body(j, _):
        pltpu.async_copy(xs2_hbm.at[srcC.at[j]], rows, sem).wait()
        pltpu.async_copy(rows, acc_sh.at[dstC.at[j]], sem, add=True).wait()
        return 0

    lax.fori_loop(0, RPT, body, 0)
    plsc.subcore_barrier()

    @pl.when(sid == 0)
    def _():
        pltpu.sync_copy(acc_sh, out2.at[cid])


def _conv2_sc(src2d, dst2d, xs2_pad):
    mesh = plsc.VectorSubcoreMesh(core_axis_name="c", subcore_axis_name="s")
    f = pl.kernel(
        _conv2_body,
        out_type=[jax.ShapeDtypeStruct((NC, NP, IN), jnp.float32)],
        mesh=mesh,
        scratch_types=[
            pltpu.MemorySpace.VMEM_SHARED((NP, IN), jnp.float32),
            pltpu.VMEM((RPT, CH), jnp.int32),
            pltpu.VMEM((RPT, CH), jnp.int32),
            pltpu.VMEM((CH, IN), jnp.float32),
            pltpu.SemaphoreType.DMA,
        ],
    )
    return f(src2d, dst2d, xs2_pad)


def _mm_body(x_ref, w_ref, o_ref):
    o_ref[...] = lax.dot_general(
        x_ref[...], w_ref[...], (((1,), (0,)), ((), ())),
        preferred_element_type=jnp.float32,
        precision=lax.Precision.HIGHEST)


def _xw_tc(x, W1):
    wo = W1.shape[1]
    return pl.pallas_call(
        _mm_body,
        grid=(N // _BR,),
        in_specs=[pl.BlockSpec((_BR, IN), lambda i: (i, 0)),
                  pl.BlockSpec((IN, wo), lambda i: (0, 0))],
        out_specs=pl.BlockSpec((_BR, wo), lambda i: (i, 0)),
        out_shape=jax.ShapeDtypeStruct((N, wo), jnp.float32),
    )(x, W1)


def _xs_body(xw_ref, deg_ref, xs_ref):
    xw = xw_ref[...]
    dinv0 = lax.rsqrt(deg_ref[:, 0:1] + 1.0)
    dinv1 = lax.rsqrt(deg_ref[:, 1:2] + 1.0)
    xs_ref[...] = jnp.concatenate([dinv0 * xw, dinv1 * xw], axis=1)


def _xs_tc(xw, deg_t):
    return pl.pallas_call(
        _xs_body,
        grid=(N // _BR,),
        in_specs=[pl.BlockSpec((_BR, OUT), lambda i: (i, 0)),
                  pl.BlockSpec((_BR, 3), lambda i: (i, 0))],
        out_specs=pl.BlockSpec((_BR, IN), lambda i: (i, 0)),
        out_shape=jax.ShapeDtypeStruct((N, IN), jnp.float32),
    )(xw, deg_t)


def _fuse_body(acc_ref, deg_ref, xw_ref, b1_ref, Wa_ref, ba_ref, qa_ref,
               W2_ref,
               h0_ref, h1_ref, attn_ref, fused_ref, fw_ref, xs2_ref):
    xw = xw_ref[...]
    b1 = b1_ref[...]
    hs = []
    ss = []
    for g in range(G):
        dinv = lax.rsqrt(deg_ref[:, g:g + 1] + 1.0)
        acc = (acc_ref[0, :, g * OUT:(g + 1) * OUT]
               + acc_ref[1, :, g * OUT:(g + 1) * OUT])
        h = jnp.maximum(dinv * acc + (dinv * dinv) * xw + b1[None, :], 0.0)
        hs.append(h)
        ha = jnp.tanh(
            lax.dot_general(h, Wa_ref[g], (((1,), (0,)), ((), ())),
                            preferred_element_type=jnp.float32,
                            precision=lax.Precision.HIGHEST)
            + ba_ref[g][None, :])
        s = lax.dot_general(ha, qa_ref[g][:, None], (((1,), (0,)), ((), ())),
                            preferred_element_type=jnp.float32,
                            precision=lax.Precision.HIGHEST)
        ss.append(s)
    m = jnp.maximum(ss[0], ss[1])
    e0 = jnp.exp(ss[0] - m)
    e1 = jnp.exp(ss[1] - m)
    tot = e0 + e1
    a0 = e0 / tot
    a1 = e1 / tot
    attn_ref[...] = jnp.concatenate([a0, a1], axis=1)
    fused = a0 * hs[0] + a1 * hs[1]
    h0_ref[...] = hs[0]
    h1_ref[...] = hs[1]
    fused_ref[...] = fused
    fw = lax.dot_general(fused, W2_ref[...], (((1,), (0,)), ((), ())),
                         preferred_element_type=jnp.float32,
                         precision=lax.Precision.HIGHEST)
    fw_ref[...] = fw
    dinv2 = lax.rsqrt(deg_ref[:, 2:3] + 1.0)
    xs2_ref[...] = dinv2 * fw


def _fuse_tc(outacc, outdeg, xw, b1, Wa, ba, qa, W2):
    nb = N // _BR
    return pl.pallas_call(
        _fuse_body,
        grid=(nb,),
        in_specs=[
            pl.BlockSpec((NC, _BR, IN), lambda i: (0, i, 0)),
            pl.BlockSpec((_BR, 3), lambda i: (i, 0)),
            pl.BlockSpec((_BR, OUT), lambda i: (i, 0)),
            pl.BlockSpec((OUT,), lambda i: (0,)),
            pl.BlockSpec((G, OUT, HID), lambda i: (0, 0, 0)),
            pl.BlockSpec((G, HID), lambda i: (0, 0)),
            pl.BlockSpec((G, HID), lambda i: (0, 0)),
            pl.BlockSpec((OUT, IN), lambda i: (0, 0)),
        ],
        out_specs=[
            pl.BlockSpec((_BR, OUT), lambda i: (i, 0)),
            pl.BlockSpec((_BR, OUT), lambda i: (i, 0)),
            pl.BlockSpec((_BR, G), lambda i: (i, 0)),
            pl.BlockSpec((_BR, OUT), lambda i: (i, 0)),
            pl.BlockSpec((_BR, IN), lambda i: (i, 0)),
            pl.BlockSpec((_BR, IN), lambda i: (i, 0)),
        ],
        out_shape=[
            jax.ShapeDtypeStruct((N, OUT), jnp.float32),
            jax.ShapeDtypeStruct((N, OUT), jnp.float32),
            jax.ShapeDtypeStruct((N, G), jnp.float32),
            jax.ShapeDtypeStruct((N, OUT), jnp.float32),
            jax.ShapeDtypeStruct((N, IN), jnp.float32),
            jax.ShapeDtypeStruct((N, IN), jnp.float32),
        ],
    )(outacc, outdeg, xw, b1, Wa, ba, qa, W2)


def _emb_body(acc2_ref, deg_ref, xs2_ref, b2_ref, emb_ref):
    dinv2 = lax.rsqrt(deg_ref[:, 2:3] + 1.0)
    acc = acc2_ref[0] + acc2_ref[1]
    emb_ref[...] = jnp.maximum(
        dinv2 * acc + dinv2 * xs2_ref[...] + b2_ref[...][None, :], 0.0)


def _emb_tc(out2, outdeg, xs2, b2):
    return pl.pallas_call(
        _emb_body,
        grid=(N // _BR,),
        in_specs=[
            pl.BlockSpec((NC, _BR, IN), lambda i: (0, i, 0)),
            pl.BlockSpec((_BR, 3), lambda i: (i, 0)),
            pl.BlockSpec((_BR, IN), lambda i: (i, 0)),
            pl.BlockSpec((IN,), lambda i: (0,)),
        ],
        out_specs=pl.BlockSpec((_BR, IN), lambda i: (i, 0)),
        out_shape=jax.ShapeDtypeStruct((N, IN), jnp.float32),
    )(out2, outdeg, xs2, b2)


def _readout_body(gn_ref, fused_ref, g_ref):
    gn = gn_ref[...]
    fused = fused_ref[...]
    vsum = lax.dot_general(gn, fused, (((1,), (0,)), ((), ())),
                           preferred_element_type=jnp.float32,
                           precision=lax.Precision.HIGHEST)
    row_sum = jnp.sum(gn, axis=1, keepdims=True)
    gg = vsum / row_sum
    nrm = jnp.sqrt(jnp.sum(gg * gg, axis=1, keepdims=True))
    g_ref[...] = gg / jnp.maximum(nrm, 1e-12)


def _readout_tc(graph_neigh, fused):
    return pl.pallas_call(
        _readout_body,
        grid=(N // _BR,),
        in_specs=[pl.BlockSpec((_BR, N), lambda i: (i, 0)),
                  pl.BlockSpec((N, OUT), lambda i: (0, 0))],
        out_specs=pl.BlockSpec((_BR, OUT), lambda i: (i, 0)),
        out_shape=jax.ShapeDtypeStruct((N, OUT), jnp.float32),
    )(graph_neigh, fused)


def kernel(x, edge_index, graph_neigh, all_edge_weights, W1, b1, W2, b2, Wa, ba, qa):
    src = edge_index[0].astype(jnp.int32)
    dst = edge_index[1].astype(jnp.int32)
    padi = jnp.full((EP - E,), N, jnp.int32)
    src2d = jnp.concatenate([src, padi]).reshape(ROWS, CH)
    dst2d = jnp.concatenate([dst, padi]).reshape(ROWS, CH)
    ew2d = jnp.concatenate(
        [all_edge_weights, jnp.zeros((G, EP - E), jnp.float32)], axis=1
    ).reshape(G, ROWS, CH)
    unit2d = jnp.concatenate(
        [jnp.ones((E,), jnp.float32), jnp.zeros((EP - E,), jnp.float32)]
    ).reshape(ROWS, CH)

    (outdeg,) = _deg_sc(dst2d, ew2d, unit2d)
    deg_t = outdeg.reshape(3, NP)[:, :N].T

    xw = _xw_tc(x, W1)
    xs_pack = _xs_tc(xw, deg_t)
    xs_pad = jnp.pad(xs_pack, ((0, NP - N), (0, 0)))

    (outacc,) = _conv1_sc(src2d, dst2d, ew2d, xs_pad)

    h0, h1, attn, fused, fw, xs2 = _fuse_tc(
        outacc[:, :N], deg_t, xw, b1, Wa, ba, qa, W2)

    xs2_pad = jnp.pad(xs2, ((0, NP - N), (0, 0)))
    (out2,) = _conv2_sc(src2d, dst2d, xs2_pad)

    emb = _emb_tc(out2[:, :N], deg_t, xs2, b2)
    g = _readout_tc(graph_neigh, fused)
    return (fused, emb, g, h0, h1, attn)


# speedup vs baseline: 10.7262x; 1.0545x over previous
import functools

import jax
import jax.numpy as jnp
from jax import lax
from jax.experimental import pallas as pl
from jax.experimental.pallas import tpu as pltpu
from jax.experimental.pallas import tpu_sc as plsc

N = 10000
E = 160000
IN = 128
OUT = 64
HID = 64
G = 2

NP = 10240
CH = 128
EP = 163840
ROWS = EP // CH
NC = 2
NS = 16
RPC = ROWS // NC
RPT = RPC // NS
RPD = ROWS // NS
NPT = NP // NS

_BR = 400


def _zero_vec(ref, nvec):
    zeros = jnp.zeros((16,), jnp.float32)

    def body(i, _):
        ref[pl.ds(i * 16, 16)] = zeros
        return 0

    lax.fori_loop(0, nvec, body, 0)


def _zero_rows(ref, nrows, width):
    zeros = jnp.zeros((16,), jnp.float32)

    def body(r, _):
        for k in range(width // 16):
            ref[r, pl.ds(k * 16, 16)] = zeros
        return 0

    lax.fori_loop(0, nrows, body, 0)


CHK = 8


def _deg_body(dst2d, ew2d, unit2d, outdeg,
              deg0_sh, deg1_sh, deg2_sh, dbuf, vbuf, zbuf, sem):
    cid = lax.axis_index("c")
    sid = lax.axis_index("s")

    _zero_vec(zbuf, NPT // 16)
    pltpu.sync_copy(zbuf, deg0_sh.at[pl.ds(sid * NPT, NPT)])
    pltpu.sync_copy(zbuf, deg1_sh.at[pl.ds(sid * NPT, NPT)])
    pltpu.sync_copy(zbuf, deg2_sh.at[pl.ds(sid * NPT, NPT)])
    plsc.subcore_barrier()

    @pl.when(cid == 0)
    def _():
        def deg_chunk(t, _):
            r0 = sid * RPD + t * CHK
            pltpu.sync_copy(dst2d.at[pl.ds(r0, CHK)], dbuf)
            pltpu.sync_copy(ew2d.at[0, pl.ds(r0, CHK)], vbuf.at[0])
            pltpu.sync_copy(ew2d.at[1, pl.ds(r0, CHK)], vbuf.at[1])
            pltpu.sync_copy(unit2d.at[pl.ds(r0, CHK)], vbuf.at[2])

            def deg_row(j, _):
                c1 = pltpu.async_copy(vbuf.at[0, j], deg0_sh.at[dbuf.at[j]],
                                      sem, add=True)
                c2 = pltpu.async_copy(vbuf.at[1, j], deg1_sh.at[dbuf.at[j]],
                                      sem, add=True)
                c3 = pltpu.async_copy(vbuf.at[2, j], deg2_sh.at[dbuf.at[j]],
                                      sem, add=True)
                c1.wait()
                c2.wait()
                c3.wait()
                return 0

            lax.fori_loop(0, CHK, deg_row, 0)
            return 0

        lax.fori_loop(0, RPD // CHK, deg_chunk, 0)

    plsc.subcore_barrier()

    @pl.when(jnp.logical_and(cid == 0, sid == 0))
    def _():
        pltpu.sync_copy(deg0_sh, outdeg.at[pl.ds(0, NP)])
        pltpu.sync_copy(deg1_sh, outdeg.at[pl.ds(NP, NP)])
        pltpu.sync_copy(deg2_sh, outdeg.at[pl.ds(2 * NP, NP)])


def _deg_sc(dst2d, ew2d, unit2d):
    mesh = plsc.VectorSubcoreMesh(core_axis_name="c", subcore_axis_name="s")
    f = pl.kernel(
        _deg_body,
        out_type=[jax.ShapeDtypeStruct((3 * NP,), jnp.float32)],
        mesh=mesh,
        scratch_types=[
            pltpu.MemorySpace.VMEM_SHARED((NP,), jnp.float32),
            pltpu.MemorySpace.VMEM_SHARED((NP,), jnp.float32),
            pltpu.MemorySpace.VMEM_SHARED((NP,), jnp.float32),
            pltpu.VMEM((CHK, CH), jnp.int32),
            pltpu.VMEM((3, CHK, CH), jnp.float32),
            pltpu.VMEM((NPT,), jnp.float32),
            pltpu.SemaphoreType.DMA,
        ],
    )
    return f(dst2d, ew2d, unit2d)


def _conv1_body(src2d, dst2d, ew2d, xs_hbm, outacc,
                acc_sh, sbuf, dbuf, vbuf, rows0, rows1,
                semg0, semg1, sems0, sems1):
    cid = lax.axis_index("c")
    sid = lax.axis_index("s")
    rows = (rows0, rows1)
    semg = (semg0, semg1)
    sems = (sems0, sems1)

    _zero_rows(rows0, CH, IN)
    for t in range(NPT // CH):
        pltpu.sync_copy(rows0, acc_sh.at[pl.ds(sid * NPT + t * CH, CH)])
    plsc.subcore_barrier()

    def conv_chunk(t, _):
        r0 = cid * RPC + sid * RPT + t * CHK
        pltpu.sync_copy(src2d.at[pl.ds(r0, CHK)], sbuf)
        pltpu.sync_copy(dst2d.at[pl.ds(r0, CHK)], dbuf)
        pltpu.sync_copy(ew2d.at[0, pl.ds(r0, CHK)], vbuf.at[0])
        pltpu.sync_copy(ew2d.at[1, pl.ds(r0, CHK)], vbuf.at[1])

        gds = [None, None]
        sds = [None] * CHK
        gds[0] = pltpu.async_copy(xs_hbm.at[sbuf.at[0]], rows[0], semg[0])
        for j in range(CHK):
            b = j % 2
            nb = (j + 1) % 2
            gds[b].wait()
            if j < CHK - 1:
                if j >= 1:
                    sds[j - 1].wait()
                gds[nb] = pltpu.async_copy(
                    xs_hbm.at[sbuf.at[j + 1]], rows[nb], semg[nb])

            rb = rows[b]

            def scale_body(gi, _, j=j, rb=rb):
                c0v = vbuf[0, j, pl.ds(gi * 16, 16)]
                c1v = vbuf[1, j, pl.ds(gi * 16, 16)]
                for u in range(16):
                    e = gi * 16 + u
                    c0 = c0v[u]
                    c1 = c1v[u]
                    for k in range(OUT // 16):
                        rb[e, pl.ds(k * 16, 16)] = (
                            rb[e, pl.ds(k * 16, 16)] * c0)
                        rb[e, pl.ds(OUT + k * 16, 16)] = (
                            rb[e, pl.ds(OUT + k * 16, 16)] * c1)
                return 0

            lax.fori_loop(0, CH // 16, scale_body, 0)

            sds[j] = pltpu.async_copy(rb, acc_sh.at[dbuf.at[j]], sems[b],
                                      add=True)
        sds[CHK - 2].wait()
        sds[CHK - 1].wait()
        return 0

    lax.fori_loop(0, RPT // CHK, conv_chunk, 0)
    plsc.subcore_barrier()

    @pl.when(sid == 0)
    def _():
        pltpu.sync_copy(acc_sh, outacc.at[cid])


def _conv1_sc(src2d, dst2d, ew2d, xs_pack):
    mesh = plsc.VectorSubcoreMesh(core_axis_name="c", subcore_axis_name="s")
    f = pl.kernel(
        _conv1_body,
        out_type=[jax.ShapeDtypeStruct((NC, NP, IN), jnp.float32)],
        mesh=mesh,
        scratch_types=[
            pltpu.MemorySpace.VMEM_SHARED((NP, IN), jnp.float32),
            pltpu.VMEM((CHK, CH), jnp.int32),
            pltpu.VMEM((CHK, CH), jnp.int32),
            pltpu.VMEM((G, CHK, CH), jnp.float32),
            pltpu.VMEM((CH, IN), jnp.float32),
            pltpu.VMEM((CH, IN), jnp.float32),
            pltpu.SemaphoreType.DMA,
            pltpu.SemaphoreType.DMA,
            pltpu.SemaphoreType.DMA,
            pltpu.SemaphoreType.DMA,
        ],
    )
    return f(src2d, dst2d, ew2d, xs_pack)


def _conv2_body(src2d, dst2d, xs2_hbm, out2,
                acc_sh, srcC, dstC, rows0, rows1,
                semg0, semg1, sems0, sems1):
    cid = lax.axis_index("c")
    sid = lax.axis_index("s")
    rows = (rows0, rows1)
    semg = (semg0, semg1)
    sems = (sems0, sems1)

    _zero_rows(rows0, CH, IN)
    for t in range(NPT // CH):
        pltpu.sync_copy(rows0, acc_sh.at[pl.ds(sid * NPT + t * CH, CH)])
    plsc.subcore_barrier()

    base = cid * RPC + sid * RPT
    pltpu.sync_copy(src2d.at[pl.ds(base, RPT)], srcC)
    pltpu.sync_copy(dst2d.at[pl.ds(base, RPT)], dstC)

    def chunk(t, _):
        j0 = t * CHK
        gds = [None, None]
        sds = [None] * CHK
        gds[0] = pltpu.async_copy(xs2_hbm.at[srcC.at[j0]], rows[0], semg[0])
        for j in range(CHK):
            b = j % 2
            nb = (j + 1) % 2
            gds[b].wait()
            if j < CHK - 1:
                if j >= 1:
                    sds[j - 1].wait()
                gds[nb] = pltpu.async_copy(
                    xs2_hbm.at[srcC.at[j0 + j + 1]], rows[nb], semg[nb])
            sds[j] = pltpu.async_copy(rows[b], acc_sh.at[dstC.at[j0 + j]],
                                      sems[b], add=True)
        sds[CHK - 2].wait()
        sds[CHK - 1].wait()
        return 0

    lax.fori_loop(0, RPT // CHK, chunk, 0)
    plsc.subcore_barrier()

    @pl.when(sid == 0)
    def _():
        pltpu.sync_copy(acc_sh, out2.at[cid])


def _conv2_sc(src2d, dst2d, xs2_pad):
    mesh = plsc.VectorSubcoreMesh(core_axis_name="c", subcore_axis_name="s")
    f = pl.kernel(
        _conv2_body,
        out_type=[jax.ShapeDtypeStruct((NC, NP, IN), jnp.float32)],
        mesh=mesh,
        scratch_types=[
            pltpu.MemorySpace.VMEM_SHARED((NP, IN), jnp.float32),
            pltpu.VMEM((RPT, CH), jnp.int32),
            pltpu.VMEM((RPT, CH), jnp.int32),
            pltpu.VMEM((CH, IN), jnp.float32),
            pltpu.VMEM((CH, IN), jnp.float32),
            pltpu.SemaphoreType.DMA,
            pltpu.SemaphoreType.DMA,
            pltpu.SemaphoreType.DMA,
            pltpu.SemaphoreType.DMA,
        ],
    )
    return f(src2d, dst2d, xs2_pad)


def _mm_body(x_ref, w_ref, o_ref):
    o_ref[...] = lax.dot_general(
        x_ref[...], w_ref[...], (((1,), (0,)), ((), ())),
        preferred_element_type=jnp.float32,
        precision=lax.Precision.HIGHEST)


def _xw_tc(x, W1):
    wo = W1.shape[1]
    return pl.pallas_call(
        _mm_body,
        grid=(N // _BR,),
        in_specs=[pl.BlockSpec((_BR, IN), lambda i: (i, 0)),
                  pl.BlockSpec((IN, wo), lambda i: (0, 0))],
        out_specs=pl.BlockSpec((_BR, wo), lambda i: (i, 0)),
        out_shape=jax.ShapeDtypeStruct((N, wo), jnp.float32),
    )(x, W1)


def _xs_body(xw_ref, deg_ref, xs_ref):
    xw = xw_ref[...]
    dinv0 = lax.rsqrt(deg_ref[:, 0:1] + 1.0)
    dinv1 = lax.rsqrt(deg_ref[:, 1:2] + 1.0)
    xs_ref[...] = jnp.concatenate([dinv0 * xw, dinv1 * xw], axis=1)


def _xs_tc(xw, deg_t):
    return pl.pallas_call(
        _xs_body,
        grid=(N // _BR,),
        in_specs=[pl.BlockSpec((_BR, OUT), lambda i: (i, 0)),
                  pl.BlockSpec((_BR, 3), lambda i: (i, 0))],
        out_specs=pl.BlockSpec((_BR, IN), lambda i: (i, 0)),
        out_shape=jax.ShapeDtypeStruct((N, IN), jnp.float32),
    )(xw, deg_t)


def _fuse_body(acc_ref, deg_ref, xw_ref, b1_ref, Wa_ref, ba_ref, qa_ref,
               W2_ref,
               h0_ref, h1_ref, attn_ref, fused_ref, fw_ref, xs2_ref):
    xw = xw_ref[...]
    b1 = b1_ref[...]
    hs = []
    ss = []
    for g in range(G):
        dinv = lax.rsqrt(deg_ref[:, g:g + 1] + 1.0)
        acc = (acc_ref[0, :, g * OUT:(g + 1) * OUT]
               + acc_ref[1, :, g * OUT:(g + 1) * OUT])
        h = jnp.maximum(dinv * acc + (dinv * dinv) * xw + b1[None, :], 0.0)
        hs.append(h)
        ha = jnp.tanh(
            lax.dot_general(h, Wa_ref[g], (((1,), (0,)), ((), ())),
                            preferred_element_type=jnp.float32,
                            precision=lax.Precision.HIGHEST)
            + ba_ref[g][None, :])
        s = lax.dot_general(ha, qa_ref[g][:, None], (((1,), (0,)), ((), ())),
                            preferred_element_type=jnp.float32,
                            precision=lax.Precision.HIGHEST)
        ss.append(s)
    m = jnp.maximum(ss[0], ss[1])
    e0 = jnp.exp(ss[0] - m)
    e1 = jnp.exp(ss[1] - m)
    tot = e0 + e1
    a0 = e0 / tot
    a1 = e1 / tot
    attn_ref[...] = jnp.concatenate([a0, a1], axis=1)
    fused = a0 * hs[0] + a1 * hs[1]
    h0_ref[...] = hs[0]
    h1_ref[...] = hs[1]
    fused_ref[...] = fused
    fw = lax.dot_general(fused, W2_ref[...], (((1,), (0,)), ((), ())),
                         preferred_element_type=jnp.float32,
                         precision=lax.Precision.HIGHEST)
    fw_ref[...] = fw
    dinv2 = lax.rsqrt(deg_ref[:, 2:3] + 1.0)
    xs2_ref[...] = dinv2 * fw


def _fuse_tc(outacc, outdeg, xw, b1, Wa, ba, qa, W2):
    nb = N // _BR
    return pl.pallas_call(
        _fuse_body,
        grid=(nb,),
        in_specs=[
            pl.BlockSpec((NC, _BR, IN), lambda i: (0, i, 0)),
            pl.BlockSpec((_BR, 3), lambda i: (i, 0)),
            pl.BlockSpec((_BR, OUT), lambda i: (i, 0)),
            pl.BlockSpec((OUT,), lambda i: (0,)),
            pl.BlockSpec((G, OUT, HID), lambda i: (0, 0, 0)),
            pl.BlockSpec((G, HID), lambda i: (0, 0)),
            pl.BlockSpec((G, HID), lambda i: (0, 0)),
            pl.BlockSpec((OUT, IN), lambda i: (0, 0)),
        ],
        out_specs=[
            pl.BlockSpec((_BR, OUT), lambda i: (i, 0)),
            pl.BlockSpec((_BR, OUT), lambda i: (i, 0)),
            pl.BlockSpec((_BR, G), lambda i: (i, 0)),
            pl.BlockSpec((_BR, OUT), lambda i: (i, 0)),
            pl.BlockSpec((_BR, IN), lambda i: (i, 0)),
            pl.BlockSpec((_BR, IN), lambda i: (i, 0)),
        ],
        out_shape=[
            jax.ShapeDtypeStruct((N, OUT), jnp.float32),
            jax.ShapeDtypeStruct((N, OUT), jnp.float32),
            jax.ShapeDtypeStruct((N, G), jnp.float32),
            jax.ShapeDtypeStruct((N, OUT), jnp.float32),
            jax.ShapeDtypeStruct((N, IN), jnp.float32),
            jax.ShapeDtypeStruct((N, IN), jnp.float32),
        ],
    )(outacc, outdeg, xw, b1, Wa, ba, qa, W2)


def _emb_body(acc2_ref, deg_ref, xs2_ref, b2_ref, emb_ref):
    dinv2 = lax.rsqrt(deg_ref[:, 2:3] + 1.0)
    acc = acc2_ref[0] + acc2_ref[1]
    emb_ref[...] = jnp.maximum(
        dinv2 * acc + dinv2 * xs2_ref[...] + b2_ref[...][None, :], 0.0)


def _emb_tc(out2, outdeg, xs2, b2):
    return pl.pallas_call(
        _emb_body,
        grid=(N // _BR,),
        in_specs=[
            pl.BlockSpec((NC, _BR, IN), lambda i: (0, i, 0)),
            pl.BlockSpec((_BR, 3), lambda i: (i, 0)),
            pl.BlockSpec((_BR, IN), lambda i: (i, 0)),
            pl.BlockSpec((IN,), lambda i: (0,)),
        ],
        out_specs=pl.BlockSpec((_BR, IN), lambda i: (i, 0)),
        out_shape=jax.ShapeDtypeStruct((N, IN), jnp.float32),
    )(out2, outdeg, xs2, b2)


def _readout_body(gn_ref, fused_ref, g_ref):
    gn = gn_ref[...]
    fused = fused_ref[...]
    vsum = lax.dot_general(gn, fused, (((1,), (0,)), ((), ())),
                           preferred_element_type=jnp.float32,
                           precision=lax.Precision.HIGHEST)
    row_sum = jnp.sum(gn, axis=1, keepdims=True)
    gg = vsum / row_sum
    nrm = jnp.sqrt(jnp.sum(gg * gg, axis=1, keepdims=True))
    g_ref[...] = gg / jnp.maximum(nrm, 1e-12)


def _readout_tc(graph_neigh, fused):
    return pl.pallas_call(
        _readout_body,
        grid=(N // _BR,),
        in_specs=[pl.BlockSpec((_BR, N), lambda i: (i, 0)),
                  pl.BlockSpec((N, OUT), lambda i: (0, 0))],
        out_specs=pl.BlockSpec((_BR, OUT), lambda i: (i, 0)),
        out_shape=jax.ShapeDtypeStruct((N, OUT), jnp.float32),
    )(graph_neigh, fused)


def kernel(x, edge_index, graph_neigh, all_edge_weights, W1, b1, W2, b2, Wa, ba, qa):
    src = edge_index[0].astype(jnp.int32)
    dst = edge_index[1].astype(jnp.int32)
    padi = jnp.full((EP - E,), N, jnp.int32)
    src2d = jnp.concatenate([src, padi]).reshape(ROWS, CH)
    dst2d = jnp.concatenate([dst, padi]).reshape(ROWS, CH)
    ew2d = jnp.concatenate(
        [all_edge_weights, jnp.zeros((G, EP - E), jnp.float32)], axis=1
    ).reshape(G, ROWS, CH)
    unit2d = jnp.concatenate(
        [jnp.ones((E,), jnp.float32), jnp.zeros((EP - E,), jnp.float32)]
    ).reshape(ROWS, CH)

    (outdeg,) = _deg_sc(dst2d, ew2d, unit2d)
    deg_t = outdeg.reshape(3, NP)[:, :N].T

    xw = _xw_tc(x, W1)
    xs_pack = _xs_tc(xw, deg_t)
    xs_pad = jnp.pad(xs_pack, ((0, NP - N), (0, 0)))

    (outacc,) = _conv1_sc(src2d, dst2d, ew2d, xs_pad)

    h0, h1, attn, fused, fw, xs2 = _fuse_tc(
        outacc[:, :N], deg_t, xw, b1, Wa, ba, qa, W2)

    xs2_pad = jnp.pad(xs2, ((0, NP - N), (0, 0)))
    (out2,) = _conv2_sc(src2d, dst2d, xs2_pad)

    emb = _emb_tc(out2[:, :N], deg_t, xs2, b2)
    g = _readout_tc(graph_neigh, fused)
    return (fused, emb, g, h0, h1, attn)


# speedup vs baseline: 11.7743x; 1.0977x over previous
import functools

import jax
import jax.numpy as jnp
from jax import lax
from jax.experimental import pallas as pl
from jax.experimental.pallas import tpu as pltpu
from jax.experimental.pallas import tpu_sc as plsc

N = 10000
E = 160000
IN = 128
OUT = 64
HID = 64
G = 2

NP = 10240
CH = 128
EP = 163840
ROWS = EP // CH
NC = 2
NS = 16
RPC = ROWS // NC
RPT = RPC // NS
RPD = ROWS // NS
NPT = NP // NS

_BR = 400


def _zero_vec(ref, nvec):
    zeros = jnp.zeros((16,), jnp.float32)

    def body(i, _):
        ref[pl.ds(i * 16, 16)] = zeros
        return 0

    lax.fori_loop(0, nvec, body, 0)


def _zero_rows(ref, nrows, width):
    zeros = jnp.zeros((16,), jnp.float32)

    def body(r, _):
        for k in range(width // 16):
            ref[r, pl.ds(k * 16, 16)] = zeros
        return 0

    lax.fori_loop(0, nrows, body, 0)


CHK = 16


def _deg_body(dst2d, ew2d, unit2d, outdeg,
              deg0_sh, deg1_sh, deg2_sh, dbuf, vbuf, zbuf, sem):
    cid = lax.axis_index("c")
    sid = lax.axis_index("s")

    _zero_vec(zbuf, NPT // 16)
    pltpu.sync_copy(zbuf, deg0_sh.at[pl.ds(sid * NPT, NPT)])
    pltpu.sync_copy(zbuf, deg1_sh.at[pl.ds(sid * NPT, NPT)])
    pltpu.sync_copy(zbuf, deg2_sh.at[pl.ds(sid * NPT, NPT)])
    plsc.subcore_barrier()

    @pl.when(cid == 0)
    def _():
        def deg_chunk(t, _):
            r0 = sid * RPD + t * CHK
            pltpu.sync_copy(dst2d.at[pl.ds(r0, CHK)], dbuf)
            pltpu.sync_copy(ew2d.at[0, pl.ds(r0, CHK)], vbuf.at[0])
            pltpu.sync_copy(ew2d.at[1, pl.ds(r0, CHK)], vbuf.at[1])
            pltpu.sync_copy(unit2d.at[pl.ds(r0, CHK)], vbuf.at[2])

            def deg_row(j, _):
                c1 = pltpu.async_copy(vbuf.at[0, j], deg0_sh.at[dbuf.at[j]],
                                      sem, add=True)
                c2 = pltpu.async_copy(vbuf.at[1, j], deg1_sh.at[dbuf.at[j]],
                                      sem, add=True)
                c3 = pltpu.async_copy(vbuf.at[2, j], deg2_sh.at[dbuf.at[j]],
                                      sem, add=True)
                c1.wait()
                c2.wait()
                c3.wait()
                return 0

            lax.fori_loop(0, CHK, deg_row, 0)
            return 0

        lax.fori_loop(0, RPD // CHK, deg_chunk, 0)

    plsc.subcore_barrier()

    @pl.when(jnp.logical_and(cid == 0, sid == 0))
    def _():
        pltpu.sync_copy(deg0_sh, outdeg.at[pl.ds(0, NP)])
        pltpu.sync_copy(deg1_sh, outdeg.at[pl.ds(NP, NP)])
        pltpu.sync_copy(deg2_sh, outdeg.at[pl.ds(2 * NP, NP)])


def _deg_sc(dst2d, ew2d, unit2d):
    mesh = plsc.VectorSubcoreMesh(core_axis_name="c", subcore_axis_name="s")
    f = pl.kernel(
        _deg_body,
        out_type=[jax.ShapeDtypeStruct((3 * NP,), jnp.float32)],
        mesh=mesh,
        scratch_types=[
            pltpu.MemorySpace.VMEM_SHARED((NP,), jnp.float32),
            pltpu.MemorySpace.VMEM_SHARED((NP,), jnp.float32),
            pltpu.MemorySpace.VMEM_SHARED((NP,), jnp.float32),
            pltpu.VMEM((CHK, CH), jnp.int32),
            pltpu.VMEM((3, CHK, CH), jnp.float32),
            pltpu.VMEM((NPT,), jnp.float32),
            pltpu.SemaphoreType.DMA,
        ],
    )
    return f(dst2d, ew2d, unit2d)


def _conv1_body(src2d, dst2d, ew2d, xs_hbm, outacc,
                acc_sh, sbuf, dbuf, vbuf, rows0, rows1,
                semg0, semg1, sems0, sems1):
    cid = lax.axis_index("c")
    sid = lax.axis_index("s")
    rows = (rows0, rows1)
    semg = (semg0, semg1)
    sems = (sems0, sems1)

    _zero_rows(rows0, CH, IN)
    for t in range(NPT // CH):
        pltpu.sync_copy(rows0, acc_sh.at[pl.ds(sid * NPT + t * CH, CH)])
    plsc.subcore_barrier()

    def conv_chunk(t, _):
        r0 = cid * RPC + sid * RPT + t * CHK
        pltpu.sync_copy(src2d.at[pl.ds(r0, CHK)], sbuf)
        pltpu.sync_copy(dst2d.at[pl.ds(r0, CHK)], dbuf)
        pltpu.sync_copy(ew2d.at[0, pl.ds(r0, CHK)], vbuf.at[0])
        pltpu.sync_copy(ew2d.at[1, pl.ds(r0, CHK)], vbuf.at[1])

        gds = [None, None]
        sds = [None] * CHK
        gds[0] = pltpu.async_copy(xs_hbm.at[sbuf.at[0]], rows[0], semg[0])
        for j in range(CHK):
            b = j % 2
            nb = (j + 1) % 2
            gds[b].wait()
            if j < CHK - 1:
                if j >= 1:
                    sds[j - 1].wait()
                gds[nb] = pltpu.async_copy(
                    xs_hbm.at[sbuf.at[j + 1]], rows[nb], semg[nb])

            rb = rows[b]

            def scale_body(gi, _, j=j, rb=rb):
                c0v = vbuf[0, j, pl.ds(gi * 16, 16)]
                c1v = vbuf[1, j, pl.ds(gi * 16, 16)]
                for u in range(16):
                    e = gi * 16 + u
                    c0 = c0v[u]
                    c1 = c1v[u]
                    for k in range(OUT // 16):
                        rb[e, pl.ds(k * 16, 16)] = (
                            rb[e, pl.ds(k * 16, 16)] * c0)
                        rb[e, pl.ds(OUT + k * 16, 16)] = (
                            rb[e, pl.ds(OUT + k * 16, 16)] * c1)
                return 0

            lax.fori_loop(0, CH // 16, scale_body, 0)

            sds[j] = pltpu.async_copy(rb, acc_sh.at[dbuf.at[j]], sems[b],
                                      add=True)
        sds[CHK - 2].wait()
        sds[CHK - 1].wait()
        return 0

    lax.fori_loop(0, RPT // CHK, conv_chunk, 0)
    plsc.subcore_barrier()

    @pl.when(sid == 0)
    def _():
        pltpu.sync_copy(acc_sh, outacc.at[cid])


def _conv1_sc(src2d, dst2d, ew2d, xs_pack):
    mesh = plsc.VectorSubcoreMesh(core_axis_name="c", subcore_axis_name="s")
    f = pl.kernel(
        _conv1_body,
        out_type=[jax.ShapeDtypeStruct((NC, NP, IN), jnp.float32)],
        mesh=mesh,
        scratch_types=[
            pltpu.MemorySpace.VMEM_SHARED((NP, IN), jnp.float32),
            pltpu.VMEM((CHK, CH), jnp.int32),
            pltpu.VMEM((CHK, CH), jnp.int32),
            pltpu.VMEM((G, CHK, CH), jnp.float32),
            pltpu.VMEM((CH, IN), jnp.float32),
            pltpu.VMEM((CH, IN), jnp.float32),
            pltpu.SemaphoreType.DMA,
            pltpu.SemaphoreType.DMA,
            pltpu.SemaphoreType.DMA,
            pltpu.SemaphoreType.DMA,
        ],
    )
    return f(src2d, dst2d, ew2d, xs_pack)


def _conv2_body(src2d, dst2d, xs2_hbm, out2,
                acc_sh, srcC, dstC, rows0, rows1,
                semg0, semg1, sems0, sems1):
    cid = lax.axis_index("c")
    sid = lax.axis_index("s")
    rows = (rows0, rows1)
    semg = (semg0, semg1)
    sems = (sems0, sems1)

    _zero_rows(rows0, CH, IN)
    for t in range(NPT // CH):
        pltpu.sync_copy(rows0, acc_sh.at[pl.ds(sid * NPT + t * CH, CH)])
    plsc.subcore_barrier()

    base = cid * RPC + sid * RPT
    pltpu.sync_copy(src2d.at[pl.ds(base, RPT)], srcC)
    pltpu.sync_copy(dst2d.at[pl.ds(base, RPT)], dstC)

    def chunk(t, _):
        j0 = t * CHK
        gds = [None, None]
        sds = [None] * CHK
        gds[0] = pltpu.async_copy(xs2_hbm.at[srcC.at[j0]], rows[0], semg[0])
        for j in range(CHK):
            b = j % 2
            nb = (j + 1) % 2
            gds[b].wait()
            if j < CHK - 1:
                if j >= 1:
                    sds[j - 1].wait()
                gds[nb] = pltpu.async_copy(
                    xs2_hbm.at[srcC.at[j0 + j + 1]], rows[nb], semg[nb])
            sds[j] = pltpu.async_copy(rows[b], acc_sh.at[dstC.at[j0 + j]],
                                      sems[b], add=True)
        sds[CHK - 2].wait()
        sds[CHK - 1].wait()
        return 0

    lax.fori_loop(0, RPT // CHK, chunk, 0)
    plsc.subcore_barrier()

    @pl.when(sid == 0)
    def _():
        pltpu.sync_copy(acc_sh, out2.at[cid])


def _conv2_sc(src2d, dst2d, xs2_pad):
    mesh = plsc.VectorSubcoreMesh(core_axis_name="c", subcore_axis_name="s")
    f = pl.kernel(
        _conv2_body,
        out_type=[jax.ShapeDtypeStruct((NC, NP, IN), jnp.float32)],
        mesh=mesh,
        scratch_types=[
            pltpu.MemorySpace.VMEM_SHARED((NP, IN), jnp.float32),
            pltpu.VMEM((RPT, CH), jnp.int32),
            pltpu.VMEM((RPT, CH), jnp.int32),
            pltpu.VMEM((CH, IN), jnp.float32),
            pltpu.VMEM((CH, IN), jnp.float32),
            pltpu.SemaphoreType.DMA,
            pltpu.SemaphoreType.DMA,
            pltpu.SemaphoreType.DMA,
            pltpu.SemaphoreType.DMA,
        ],
    )
    return f(src2d, dst2d, xs2_pad)


def _mm_body(x_ref, w_ref, o_ref):
    o_ref[...] = lax.dot_general(
        x_ref[...], w_ref[...], (((1,), (0,)), ((), ())),
        preferred_element_type=jnp.float32,
        precision=lax.Precision.HIGHEST)


def _xw_tc(x, W1):
    wo = W1.shape[1]
    return pl.pallas_call(
        _mm_body,
        grid=(N // _BR,),
        in_specs=[pl.BlockSpec((_BR, IN), lambda i: (i, 0)),
                  pl.BlockSpec((IN, wo), lambda i: (0, 0))],
        out_specs=pl.BlockSpec((_BR, wo), lambda i: (i, 0)),
        out_shape=jax.ShapeDtypeStruct((N, wo), jnp.float32),
    )(x, W1)


def _xs_body(xw_ref, deg_ref, xs_ref):
    xw = xw_ref[...]
    dinv0 = lax.rsqrt(deg_ref[:, 0:1] + 1.0)
    dinv1 = lax.rsqrt(deg_ref[:, 1:2] + 1.0)
    xs_ref[...] = jnp.concatenate([dinv0 * xw, dinv1 * xw], axis=1)


def _xs_tc(xw, deg_t):
    return pl.pallas_call(
        _xs_body,
        grid=(N // _BR,),
        in_specs=[pl.BlockSpec((_BR, OUT), lambda i: (i, 0)),
                  pl.BlockSpec((_BR, 3), lambda i: (i, 0))],
        out_specs=pl.BlockSpec((_BR, IN), lambda i: (i, 0)),
        out_shape=jax.ShapeDtypeStruct((N, IN), jnp.float32),
    )(xw, deg_t)


def _fuse_body(acc_ref, deg_ref, xw_ref, b1_ref, Wa_ref, ba_ref, qa_ref,
               W2_ref,
               h0_ref, h1_ref, attn_ref, fused_ref, fw_ref, xs2_ref):
    xw = xw_ref[...]
    b1 = b1_ref[...]
    hs = []
    ss = []
    for g in range(G):
        dinv = lax.rsqrt(deg_ref[:, g:g + 1] + 1.0)
        acc = (acc_ref[0, :, g * OUT:(g + 1) * OUT]
               + acc_ref[1, :, g * OUT:(g + 1) * OUT])
        h = jnp.maximum(dinv * acc + (dinv * dinv) * xw + b1[None, :], 0.0)
        hs.append(h)
        ha = jnp.tanh(
            lax.dot_general(h, Wa_ref[g], (((1,), (0,)), ((), ())),
                            preferred_element_type=jnp.float32,
                            precision=lax.Precision.HIGHEST)
            + ba_ref[g][None, :])
        s = lax.dot_general(ha, qa_ref[g][:, None], (((1,), (0,)), ((), ())),
                            preferred_element_type=jnp.float32,
                            precision=lax.Precision.HIGHEST)
        ss.append(s)
    m = jnp.maximum(ss[0], ss[1])
    e0 = jnp.exp(ss[0] - m)
    e1 = jnp.exp(ss[1] - m)
    tot = e0 + e1
    a0 = e0 / tot
    a1 = e1 / tot
    attn_ref[...] = jnp.concatenate([a0, a1], axis=1)
    fused = a0 * hs[0] + a1 * hs[1]
    h0_ref[...] = hs[0]
    h1_ref[...] = hs[1]
    fused_ref[...] = fused
    fw = lax.dot_general(fused, W2_ref[...], (((1,), (0,)), ((), ())),
                         preferred_element_type=jnp.float32,
                         precision=lax.Precision.HIGHEST)
    fw_ref[...] = fw
    dinv2 = lax.rsqrt(deg_ref[:, 2:3] + 1.0)
    xs2_ref[...] = dinv2 * fw


def _fuse_tc(outacc, outdeg, xw, b1, Wa, ba, qa, W2):
    nb = N // _BR
    return pl.pallas_call(
        _fuse_body,
        grid=(nb,),
        in_specs=[
            pl.BlockSpec((NC, _BR, IN), lambda i: (0, i, 0)),
            pl.BlockSpec((_BR, 3), lambda i: (i, 0)),
            pl.BlockSpec((_BR, OUT), lambda i: (i, 0)),
            pl.BlockSpec((OUT,), lambda i: (0,)),
            pl.BlockSpec((G, OUT, HID), lambda i: (0, 0, 0)),
            pl.BlockSpec((G, HID), lambda i: (0, 0)),
            pl.BlockSpec((G, HID), lambda i: (0, 0)),
            pl.BlockSpec((OUT, IN), lambda i: (0, 0)),
        ],
        out_specs=[
            pl.BlockSpec((_BR, OUT), lambda i: (i, 0)),
            pl.BlockSpec((_BR, OUT), lambda i: (i, 0)),
            pl.BlockSpec((_BR, G), lambda i: (i, 0)),
            pl.BlockSpec((_BR, OUT), lambda i: (i, 0)),
            pl.BlockSpec((_BR, IN), lambda i: (i, 0)),
            pl.BlockSpec((_BR, IN), lambda i: (i, 0)),
        ],
        out_shape=[
            jax.ShapeDtypeStruct((N, OUT), jnp.float32),
            jax.ShapeDtypeStruct((N, OUT), jnp.float32),
            jax.ShapeDtypeStruct((N, G), jnp.float32),
            jax.ShapeDtypeStruct((N, OUT), jnp.float32),
            jax.ShapeDtypeStruct((N, IN), jnp.float32),
            jax.ShapeDtypeStruct((N, IN), jnp.float32),
        ],
    )(outacc, outdeg, xw, b1, Wa, ba, qa, W2)


def _emb_body(acc2_ref, deg_ref, xs2_ref, b2_ref, emb_ref):
    dinv2 = lax.rsqrt(deg_ref[:, 2:3] + 1.0)
    acc = acc2_ref[0] + acc2_ref[1]
    emb_ref[...] = jnp.maximum(
        dinv2 * acc + dinv2 * xs2_ref[...] + b2_ref[...][None, :], 0.0)


def _emb_tc(out2, outdeg, xs2, b2):
    return pl.pallas_call(
        _emb_body,
        grid=(N // _BR,),
        in_specs=[
            pl.BlockSpec((NC, _BR, IN), lambda i: (0, i, 0)),
            pl.BlockSpec((_BR, 3), lambda i: (i, 0)),
            pl.BlockSpec((_BR, IN), lambda i: (i, 0)),
            pl.BlockSpec((IN,), lambda i: (0,)),
        ],
        out_specs=pl.BlockSpec((_BR, IN), lambda i: (i, 0)),
        out_shape=jax.ShapeDtypeStruct((N, IN), jnp.float32),
    )(out2, outdeg, xs2, b2)


def _readout_body(gn_ref, fused_ref, g_ref):
    gn = gn_ref[...]
    fused = fused_ref[...]
    vsum = lax.dot_general(gn, fused, (((1,), (0,)), ((), ())),
                           preferred_element_type=jnp.float32,
                           precision=lax.Precision.HIGHEST)
    row_sum = jnp.sum(gn, axis=1, keepdims=True)
    gg = vsum / row_sum
    nrm = jnp.sqrt(jnp.sum(gg * gg, axis=1, keepdims=True))
    g_ref[...] = gg / jnp.maximum(nrm, 1e-12)


def _readout_tc(graph_neigh, fused):
    return pl.pallas_call(
        _readout_body,
        grid=(N // _BR,),
        in_specs=[pl.BlockSpec((_BR, N), lambda i: (i, 0)),
                  pl.BlockSpec((N, OUT), lambda i: (0, 0))],
        out_specs=pl.BlockSpec((_BR, OUT), lambda i: (i, 0)),
        out_shape=jax.ShapeDtypeStruct((N, OUT), jnp.float32),
    )(graph_neigh, fused)


def kernel(x, edge_index, graph_neigh, all_edge_weights, W1, b1, W2, b2, Wa, ba, qa):
    src = edge_index[0].astype(jnp.int32)
    dst = edge_index[1].astype(jnp.int32)
    padi = jnp.full((EP - E,), N, jnp.int32)
    src2d = jnp.concatenate([src, padi]).reshape(ROWS, CH)
    dst2d = jnp.concatenate([dst, padi]).reshape(ROWS, CH)
    ew2d = jnp.concatenate(
        [all_edge_weights, jnp.zeros((G, EP - E), jnp.float32)], axis=1
    ).reshape(G, ROWS, CH)
    unit2d = jnp.concatenate(
        [jnp.ones((E,), jnp.float32), jnp.zeros((EP - E,), jnp.float32)]
    ).reshape(ROWS, CH)

    (outdeg,) = _deg_sc(dst2d, ew2d, unit2d)
    deg_t = outdeg.reshape(3, NP)[:, :N].T

    xw = _xw_tc(x, W1)
    xs_pack = _xs_tc(xw, deg_t)
    xs_pad = jnp.pad(xs_pack, ((0, NP - N), (0, 0)))

    (outacc,) = _conv1_sc(src2d, dst2d, ew2d, xs_pad)

    h0, h1, attn, fused, fw, xs2 = _fuse_tc(
        outacc[:, :N], deg_t, xw, b1, Wa, ba, qa, W2)

    xs2_pad = jnp.pad(xs2, ((0, NP - N), (0, 0)))
    (out2,) = _conv2_sc(src2d, dst2d, xs2_pad)

    emb = _emb_tc(out2[:, :N], deg_t, xs2, b2)
    g = _readout_tc(graph_neigh, fused)
    return (fused, emb, g, h0, h1, attn)
